# Initial kernel scaffold; baseline (speedup 1.0000x reference)
import jax
import jax.numpy as jnp
from jax.experimental import pallas as pl


def kernel(x, W):
    raise NotImplementedError("write your pallas kernel here")


# speedup vs baseline: 1.0753x; 1.0753x over previous
import functools

import jax
import jax.numpy as jnp
from jax import lax
from jax.experimental import pallas as pl
from jax.experimental.pallas import tpu as pltpu
from jax.experimental.pallas import tpu_sc as plsc

EMBED = 32
NCODES = 8192
ROWS = 16 * 1024
ROW_BLK = 512
N_ROW_BLKS = ROWS // ROW_BLK
COL_BLK = 1024
N_COL_BLKS = NCODES // COL_BLK
CCOST = 0.25


def _argmin_body(x_ref, wt_ref, sw_ref, idx_ref):
    xt = x_ref[...]
    xb = xt.astype(jnp.bfloat16)
    sx = jnp.sum(xt * xt, axis=1, keepdims=True)
    best_d = jnp.full((ROW_BLK,), jnp.inf, jnp.float32)
    best_i = jnp.zeros((ROW_BLK,), jnp.int32)
    for c in range(N_COL_BLKS):
        wt = wt_ref[:, c * COL_BLK:(c + 1) * COL_BLK]
        sw = sw_ref[:, c * COL_BLK:(c + 1) * COL_BLK]
        sc = lax.dot_general(xb, wt,
                             (((1,), (0,)), ((), ())),
                             preferred_element_type=jnp.float32)
        d = (sx + sw) - 2.0 * sc
        m = jnp.min(d, axis=1)
        ii = lax.broadcasted_iota(jnp.int32, (ROW_BLK, COL_BLK), 1) + c * COL_BLK
        li = jnp.min(jnp.where(d == m[:, None], ii, jnp.int32(2 ** 30)), axis=1)
        upd = m < best_d
        best_i = jnp.where(upd, li, best_i)
        best_d = jnp.where(upd, m, best_d)
    idx_ref[0, 0, :] = best_i


def _run_argmin(x2d, wt, sw):
    idx3 = pl.pallas_call(
        _argmin_body,
        grid=(N_ROW_BLKS,),
        in_specs=[
            pl.BlockSpec((ROW_BLK, EMBED), lambda i: (i, 0)),
            pl.BlockSpec((EMBED, NCODES), lambda i: (0, 0)),
            pl.BlockSpec((1, NCODES), lambda i: (0, 0)),
        ],
        out_specs=pl.BlockSpec((1, 1, ROW_BLK), lambda i: (i, 0, 0)),
        out_shape=jax.ShapeDtypeStruct((N_ROW_BLKS, 1, ROW_BLK), jnp.int32),
    )(x2d, wt, sw)
    return idx3.reshape(ROWS)


_NC = 2
_NW = 32
_B_PER_W = ROWS // _NW
_GATHER_W = 128


@functools.cache
def _make_gather_rows():
    @functools.partial(
        pl.kernel,
        mesh=plsc.VectorSubcoreMesh(core_axis_name="c", subcore_axis_name="s"),
        out_type=jax.ShapeDtypeStruct((ROWS, _GATHER_W), jnp.float32),
        scratch_types=[
            pltpu.VMEM((_B_PER_W,), jnp.int32),
            pltpu.VMEM((_B_PER_W, _GATHER_W), jnp.float32),
            pltpu.SemaphoreType.DMA,
        ],
    )
    def _gather_rows(w_hbm, idx_hbm, out_hbm, idx_v, rows_v, sem):
        wid = lax.axis_index("s") * _NC + lax.axis_index("c")
        base = wid * _B_PER_W
        pltpu.sync_copy(idx_hbm.at[pl.ds(base, _B_PER_W)], idx_v)
        pltpu.async_copy(w_hbm.at[idx_v], rows_v, sem).wait()
        pltpu.sync_copy(rows_v, out_hbm.at[pl.ds(base, _B_PER_W)])

    return _gather_rows


def _st_loss_body(x_ref, q_ref, out_ref, loss_ref):
    i = pl.program_id(0)
    x = x_ref[...]
    q = q_ref[:, :EMBED]
    d = q - x
    out_ref[...] = x + d

    @pl.when(i == 0)
    def _init():
        loss_ref[...] = jnp.zeros((1, 1), jnp.float32)

    loss_ref[...] += jnp.sum(d * d).reshape(1, 1)


def _run_st_loss(x2d, q2d):
    return pl.pallas_call(
        _st_loss_body,
        grid=(N_ROW_BLKS,),
        in_specs=[
            pl.BlockSpec((ROW_BLK, EMBED), lambda i: (i, 0)),
            pl.BlockSpec((ROW_BLK, _GATHER_W), lambda i: (i, 0)),
        ],
        out_specs=[
            pl.BlockSpec((ROW_BLK, EMBED), lambda i: (i, 0)),
            pl.BlockSpec((1, 1), lambda i: (0, 0)),
        ],
        out_shape=[
            jax.ShapeDtypeStruct((ROWS, EMBED), jnp.float32),
            jax.ShapeDtypeStruct((1, 1), jnp.float32),
        ],
    )(x2d, q2d)


def kernel(x, W):
    x2d = x.reshape(ROWS, EMBED)
    sw = jnp.sum(W ** 2, axis=1).reshape(1, NCODES)
    idx = _run_argmin(x2d, W.T, sw)
    w_pad = jnp.pad(W, ((0, 0), (0, _GATHER_W - EMBED)))
    q128 = _make_gather_rows()(w_pad, idx)
    qst, loss_sum = _run_st_loss(x2d, q128)
    m = loss_sum[0, 0] / jnp.float32(ROWS * EMBED)
    loss = m + jnp.float32(CCOST) * m
    return qst.reshape(x.shape), loss
